# Initial kernel scaffold; baseline (speedup 1.0000x reference)
import jax
import jax.numpy as jnp
from jax.experimental import pallas as pl


def kernel(inp, skip, wt_mat, w1_mats, w2_mats, w3_mat, g1, b1, g2, b2, g3, b3):
    raise NotImplementedError("write your pallas kernel here")


# speedup vs baseline: 4.6656x; 4.6656x over previous
import functools

import jax
import jax.numpy as jnp
from jax.experimental import pallas as pl
from jax.experimental.pallas import tpu as pltpu

IN_EPS = 1e-5
NEG_SLOPE = 0.01


def _instance_norm(y, gamma, beta):
    mu = jnp.mean(y, axis=-1, keepdims=True)
    var = jnp.mean((y - mu) ** 2, axis=-1, keepdims=True)
    return (y - mu) * jax.lax.rsqrt(var + IN_EPS) * gamma + beta


def _leaky_relu(y):
    return jnp.where(y > 0, y, NEG_SLOPE * y)


def _rep(a):
    return pl.BlockSpec(a.shape, lambda b, _n=a.ndim: (0,) * _n)


def _tconv_body(x_ref, w_ref, o_ref):
    o_ref[0] = jnp.dot(w_ref[...], x_ref[0],
                       preferred_element_type=jnp.float32)


def _tconv(x_cf, wt_mat):
    B, Cin, S1 = x_cf.shape
    R = wt_mat.shape[0]
    return pl.pallas_call(
        _tconv_body,
        out_shape=jax.ShapeDtypeStruct((B, R, S1), jnp.float32),
        grid=(B,),
        in_specs=[pl.BlockSpec((1, Cin, S1), lambda b: (b, 0, 0)), _rep(wt_mat)],
        out_specs=pl.BlockSpec((1, R, S1), lambda b: (b, 0, 0)),
        compiler_params=pltpu.CompilerParams(dimension_semantics=("parallel",)),
    )(x_cf, wt_mat)


def _taps_to_scratch(src, p_s, masks, offs, rows):
    s = src.shape[1]
    for t, (off, mask) in enumerate(zip(offs, masks)):
        shifted = pltpu.roll(src, (-off) % s, axis=1) if off else src
        p_s[t * rows:(t + 1) * rows, :] = (shifted * mask).astype(p_s.dtype)


def _conv3_from_scratch(p_s, w_ref, cout, hw, s):
    sm = s - hw
    acc = jnp.dot(w_ref[1], p_s[...], preferred_element_type=jnp.float32)
    d0 = jnp.dot(w_ref[0], p_s[:, :sm], preferred_element_type=jnp.float32)
    acc = acc + jnp.concatenate(
        [jnp.zeros((cout, hw), jnp.float32), d0], axis=1)
    d2 = jnp.dot(w_ref[2], p_s[:, hw:], preferred_element_type=jnp.float32)
    return acc + jnp.concatenate(
        [d2, jnp.zeros((cout, hw), jnp.float32)], axis=1)


def _fused_body(x_ref, w1_ref, g1_ref, b1_ref, w3_ref, g3_ref, b3_ref,
                w2_ref, g2_ref, b2_ref, o_ref, p1_s, p2_s, *, hw, s, wlen, hlen):
    cin = x_ref.shape[1]
    cout = o_ref.shape[1]
    x = x_ref[0]

    lane = jax.lax.broadcasted_iota(jnp.int32, (1, s), 1)
    hwi = lane % hw
    hv = hwi // wlen
    wv = hwi % wlen
    offs, masks = [], []
    for oh in (-1, 0, 1):
        for ow in (-1, 0, 1):
            offs.append(oh * wlen + ow)
            valid = ((hv + oh >= 0) & (hv + oh < hlen)
                     & (wv + ow >= 0) & (wv + ow < wlen))
            masks.append(valid.astype(jnp.float32))

    _taps_to_scratch(x, p1_s, masks, offs, cin)
    y1 = _leaky_relu(_instance_norm(
        _conv3_from_scratch(p1_s, w1_ref, cout, hw, s), g1_ref[...], b1_ref[...]))

    r = _instance_norm(
        jnp.dot(w3_ref[...], x, preferred_element_type=jnp.float32),
        g3_ref[...], b3_ref[...])

    _taps_to_scratch(y1, p2_s, masks, offs, cout)
    y2 = _instance_norm(
        _conv3_from_scratch(p2_s, w2_ref, cout, hw, s), g2_ref[...], b2_ref[...])
    o_ref[0] = _leaky_relu(y2 + r)


def _fused(x, w1b, g1, b1, w3_mat, g3, b3, w2b, g2, b2, *, hw, s, wlen, hlen):
    B, cin, S = x.shape
    cout = w1b.shape[1]
    body = functools.partial(_fused_body, hw=hw, s=s, wlen=wlen, hlen=hlen)
    return pl.pallas_call(
        body,
        out_shape=jax.ShapeDtypeStruct((B, cout, S), jnp.float32),
        grid=(B,),
        in_specs=[pl.BlockSpec((1, cin, S), lambda b: (b, 0, 0)),
                  _rep(w1b), _rep(g1), _rep(b1),
                  _rep(w3_mat), _rep(g3), _rep(b3),
                  _rep(w2b), _rep(g2), _rep(b2)],
        out_specs=pl.BlockSpec((1, cout, S), lambda b: (b, 0, 0)),
        scratch_shapes=[pltpu.VMEM((9 * cin, S), jnp.bfloat16),
                        pltpu.VMEM((9 * cout, S), jnp.bfloat16)],
        compiler_params=pltpu.CompilerParams(
            dimension_semantics=("parallel",),
            vmem_limit_bytes=48 * 1024 * 1024),
    )(x, w1b, g1, b1, w3_mat, g3, b3, w2b, g2, b2)


def kernel(inp, skip, wt_mat, w1_mats, w2_mats, w3_mat, g1, b1, g2, b2, g3, b3):
    B, Cin, D, H, W = inp.shape
    Cout = skip.shape[1]
    up_k = 2
    Do, Ho, Wo = D * up_k, H * up_k, W * up_k
    HWo = Ho * Wo
    S = Do * HWo

    up = _tconv(inp.reshape(B, Cin, D * H * W), wt_mat)

    up = up.reshape(B, Cout, up_k, up_k, up_k, D, H, W)
    up = up.transpose(0, 1, 5, 2, 6, 3, 7, 4).reshape(B, Cout, S)
    x = jnp.concatenate([up, skip.reshape(B, Cout, S)], axis=1)

    out = _fused(x, w1_mats.astype(jnp.bfloat16), g1, b1,
                 w3_mat, g3, b3,
                 w2_mats.astype(jnp.bfloat16), g2, b2,
                 hw=HWo, s=S, wlen=Wo, hlen=Ho)
    return out.reshape(B, Cout, Do, Ho, Wo)


# speedup vs baseline: 6.3511x; 1.3613x over previous
import functools

import jax
import jax.numpy as jnp
from jax.experimental import pallas as pl
from jax.experimental.pallas import tpu as pltpu

IN_EPS = 1e-5
NEG_SLOPE = 0.01


def _instance_norm(y, gamma, beta):
    mu = jnp.mean(y, axis=-1, keepdims=True)
    var = jnp.mean((y - mu) ** 2, axis=-1, keepdims=True)
    return (y - mu) * jax.lax.rsqrt(var + IN_EPS) * gamma + beta


def _leaky_relu(y):
    return jnp.where(y > 0, y, NEG_SLOPE * y)


def _rep(a):
    return pl.BlockSpec(a.shape, lambda b, _n=a.ndim: (0,) * _n)


def _hw_masks(seg, hlen, wlen):
    l = jax.lax.broadcasted_iota(jnp.int32, (1, seg), 1)
    hw = l % (hlen * wlen)
    hv = hw // wlen
    wv = hw % wlen
    masks = {}
    for sh in (-1, 0, 1):
        for sw in (-1, 0, 1):
            if sh == 0 and sw == 0:
                masks[(sh, sw)] = None
                continue
            valid = ((hv + sh >= 0) & (hv + sh < hlen)
                     & (wv + sw >= 0) & (wv + sw < wlen))
            masks[(sh, sw)] = valid.astype(jnp.float32)
    return masks


def _taps_to_scratch(x, p_s, masks, *, seg, pad, wlen, rows):
    segp = seg + 2 * pad
    for kh in range(3):
        for kw in range(3):
            t = kh * 3 + kw
            for qh in range(2):
                for qw in range(2):
                    q = qh * 2 + qw
                    qsrc = ((qh + kh - 1) % 2) * 2 + ((qw + kw - 1) % 2)
                    sh = (qh + kh - 1) // 2
                    sw = (qw + kw - 1) // 2
                    src = x[:, qsrc * seg:(qsrc + 1) * seg]
                    off = sh * wlen + sw
                    if off:
                        src = pltpu.roll(src, (-off) % seg, axis=1)
                    m = masks[(sh, sw)]
                    if m is not None:
                        src = src * m
                    c0 = q * segp + pad
                    p_s[t * rows:(t + 1) * rows, c0:c0 + seg] = (
                        src.astype(p_s.dtype))


def _conv3_hybrid(p_s, w_ref, *, seg, pad):
    segp = seg + 2 * pad
    outs = []
    for q in range(4):
        acc = None
        for kd in range(3):
            c0 = q * segp + kd * pad
            d = jnp.dot(w_ref[kd], p_s[:, c0:c0 + seg],
                        preferred_element_type=jnp.float32)
            acc = d if acc is None else acc + d
        outs.append(acc)
    return jnp.concatenate(outs, axis=1)


def _zero_pads(p_s, *, seg, pad):
    segp = seg + 2 * pad
    z = jnp.zeros((p_s.shape[0], pad), p_s.dtype)
    for q in range(4):
        p_s[:, q * segp:q * segp + pad] = z
        p_s[:, q * segp + pad + seg:(q + 1) * segp] = z


def _fused_body(inp_ref, skip_ref, wt_ref, w1_ref, g1_ref, b1_ref,
                w3_ref, g3_ref, b3_ref, w2_ref, g2_ref, b2_ref,
                o_ref, p1_s, p2_s, *, seg, pad, wlen, hlen):
    cout = o_ref.shape[1]
    dblk = pad
    nd = seg // dblk

    up2 = jnp.dot(wt_ref[...], inp_ref[0],
                  preferred_element_type=jnp.float32)

    segs = []
    for q in range(4):
        a = up2[q * cout:(q + 1) * cout]
        b = up2[(4 + q) * cout:(5 + q) * cout]
        for dp in range(nd // 2):
            segs.append(a[:, dp * dblk:(dp + 1) * dblk])
            segs.append(b[:, dp * dblk:(dp + 1) * dblk])
    x_up = jnp.concatenate(segs, axis=1)
    x = jnp.concatenate([x_up, skip_ref[0]], axis=0)

    masks = _hw_masks(seg, hlen, wlen)

    _zero_pads(p1_s, seg=seg, pad=pad)
    _taps_to_scratch(x, p1_s, masks, seg=seg, pad=pad, wlen=wlen,
                     rows=2 * cout)
    y1 = _leaky_relu(_instance_norm(
        _conv3_hybrid(p1_s, w1_ref, seg=seg, pad=pad),
        g1_ref[...], b1_ref[...]))

    r = _instance_norm(
        jnp.dot(w3_ref[...], x, preferred_element_type=jnp.float32),
        g3_ref[...], b3_ref[...])

    _zero_pads(p2_s, seg=seg, pad=pad)
    _taps_to_scratch(y1, p2_s, masks, seg=seg, pad=pad, wlen=wlen, rows=cout)
    y2 = _instance_norm(
        _conv3_hybrid(p2_s, w2_ref, seg=seg, pad=pad),
        g2_ref[...], b2_ref[...])
    o_ref[0] = _leaky_relu(y2 + r)


def kernel(inp, skip, wt_mat, w1_mats, w2_mats, w3_mat, g1, b1, g2, b2, g3, b3):
    B, Cin, D, H, W = inp.shape
    Cout = skip.shape[1]
    Do, Ho, Wo = 2 * D, 2 * H, 2 * W
    S = Do * Ho * Wo
    seg = Do * H * W
    pad = H * W

    wt2 = wt_mat.reshape(Cout, 8, Cin).transpose(1, 0, 2).reshape(8 * Cout, Cin)

    skip_h = (skip.reshape(B, Cout, Do, H, 2, W, 2)
              .transpose(0, 1, 4, 6, 2, 3, 5).reshape(B, Cout, S))

    body = functools.partial(_fused_body, seg=seg, pad=pad, wlen=W, hlen=H)
    out = pl.pallas_call(
        body,
        out_shape=jax.ShapeDtypeStruct((B, Cout, S), jnp.float32),
        grid=(B,),
        in_specs=[pl.BlockSpec((1, Cin, D * H * W), lambda b: (b, 0, 0)),
                  pl.BlockSpec((1, Cout, S), lambda b: (b, 0, 0)),
                  _rep(wt2),
                  _rep(w1_mats), _rep(g1), _rep(b1),
                  _rep(w3_mat), _rep(g3), _rep(b3),
                  _rep(w2_mats), _rep(g2), _rep(b2)],
        out_specs=pl.BlockSpec((1, Cout, S), lambda b: (b, 0, 0)),
        scratch_shapes=[
            pltpu.VMEM((9 * 2 * Cout, 4 * (seg + 2 * pad)), jnp.bfloat16),
            pltpu.VMEM((9 * Cout, 4 * (seg + 2 * pad)), jnp.bfloat16)],
        compiler_params=pltpu.CompilerParams(
            dimension_semantics=("parallel",),
            vmem_limit_bytes=48 * 1024 * 1024),
    )(inp.reshape(B, Cin, D * H * W), skip_h, wt2,
      w1_mats.astype(jnp.bfloat16), g1, b1, w3_mat, g3, b3,
      w2_mats.astype(jnp.bfloat16), g2, b2)

    out = (out.reshape(B, Cout, 2, 2, Do, H, W)
           .transpose(0, 1, 4, 5, 2, 6, 3).reshape(B, Cout, Do, Ho, Wo))
    return out
